# baseline (speedup 1.0000x reference)
import functools

import jax
import jax.numpy as jnp
from jax import lax
from jax.experimental import pallas as pl
from jax.experimental.pallas import tpu as pltpu
from jax.experimental.pallas import tpu_sc as plsc

D = 64
H = 128
NC, NS = 2, 16
NW = NC * NS
K = 128


def _sc_gather_add(ids, cats, itab, ctab, b_size):
    n_rows = ids.shape[0]
    rows_per_w = n_rows // NW
    n_chunks = rows_per_w // K
    b_mask = b_size - 1
    mesh = plsc.VectorSubcoreMesh(
        core_axis_name="c", subcore_axis_name="s", num_cores=NC, num_subcores=NS
    )

    @functools.partial(
        pl.kernel,
        out_type=jax.ShapeDtypeStruct((n_rows, D), jnp.float32),
        mesh=mesh,
        scratch_types=[
            pltpu.VMEM((2, K), jnp.int32),
            pltpu.VMEM((2, K), jnp.int32),
            pltpu.VMEM((2, K), jnp.int32),
            pltpu.VMEM((2, K, D), jnp.float32),
            pltpu.VMEM((2, K, D), jnp.float32),
            pltpu.VMEM((2, K, D), jnp.float32),
            pltpu.SemaphoreType.DMA,
            pltpu.SemaphoreType.DMA,
            pltpu.SemaphoreType.DMA,
            pltpu.SemaphoreType.DMA,
            pltpu.SemaphoreType.DMA,
            pltpu.SemaphoreType.DMA,
        ],
        compiler_params=pltpu.CompilerParams(use_tc_tiling_on_sc=False),
    )
    def sc_kernel(ids_hbm, cats_hbm, itab_hbm, ctab_hbm, out_hbm,
                  idx_i, idx_c, dst_i, ebuf, cbuf, obuf,
                  sgi0, sgc0, so0, sgi1, sgc1, so1):
        sgi = (sgi0, sgi1)
        sgc = (sgc0, sgc1)
        so = (so0, so1)
        wid = lax.axis_index("s") * NC + lax.axis_index("c")
        base = wid * rows_per_w

        def stage_idx(c, s):
            off = base + c * K
            pltpu.sync_copy(ids_hbm.at[pl.ds(off, K)], idx_i.at[s])
            pltpu.sync_copy(cats_hbm.at[pl.ds(off, K)], idx_c.at[s])

        def issue_gather(s):
            pltpu.async_copy(itab_hbm.at[idx_i.at[s]], ebuf.at[s], sgi[s])
            pltpu.async_copy(ctab_hbm.at[idx_c.at[s]], cbuf.at[s], sgc[s])

        def wait_gather(s):
            pltpu.make_async_copy(itab_hbm.at[idx_i.at[s]], ebuf.at[s], sgi[s]).wait()
            pltpu.make_async_copy(ctab_hbm.at[idx_c.at[s]], cbuf.at[s], sgc[s]).wait()

        def compute_dst(c, s):
            off = base + c * K

            def vec(v, carry):
                pos = off + v * 16 + lax.iota(jnp.int32, 16)
                b = lax.bitwise_and(pos, b_mask)
                dst = (
                    lax.bitwise_and(pos, ~b_mask)
                    + lax.bitwise_and(b, ~511)
                    + lax.shift_left(lax.bitwise_and(b, 255), 1)
                    + lax.bitwise_and(lax.shift_right_logical(b, 8), 1)
                )
                dst_i[s, pl.ds(v * 16, 16)] = dst
                return carry

            lax.fori_loop(0, K // 16, vec, 0)

        def do_add(s):
            def add_row(j, carry):
                for t in range(D // 16):
                    sl = pl.ds(t * 16, 16)
                    obuf[s, j, sl] = ebuf[s, j, sl] + cbuf[s, j, sl]
                return carry
            lax.fori_loop(0, K, add_row, 0)

        def issue_out(s):
            pltpu.async_copy(obuf.at[s], out_hbm.at[dst_i.at[s]], so[s])

        def wait_out(s):
            pltpu.make_async_copy(obuf.at[s], out_hbm.at[dst_i.at[s]], so[s]).wait()

        stage_idx(0, 0)
        issue_gather(0)
        stage_idx(1, 1)
        issue_gather(1)

        def body(g2, carry):
            for s in (0, 1):
                c = 2 * g2 + s
                wait_gather(s)

                @pl.when(g2 > 0)
                def _():
                    wait_out(s)

                compute_dst(c, s)
                do_add(s)
                issue_out(s)

                @pl.when(c + 2 < n_chunks)
                def _():
                    stage_idx(c + 2, s)
                    issue_gather(s)
            return carry

        lax.fori_loop(0, n_chunks // 2, body, 0)
        wait_out(0)
        wait_out(1)

    return sc_kernel(ids, cats, itab, ctab)


def _tc_mlp_normalize(e2, W1d, b1d, W2d, b2d, n_b, n_c):
    blkj = 2048
    n_k = n_b // (2 * blkj)

    def body(e_ref, w1_ref, b1_ref, w2_ref, b2_ref, o_ref):
        ev = e_ref[...]
        h = jnp.dot(
            ev.astype(jnp.bfloat16), w1_ref[...],
            preferred_element_type=jnp.float32,
        )
        h = jnp.maximum(h + b1_ref[...], 0.0)
        r = jnp.dot(
            h.astype(jnp.bfloat16), w2_ref[...],
            preferred_element_type=jnp.float32,
        )
        r = r + b2_ref[...] + ev
        s = r * r
        col = lax.broadcasted_iota(jnp.int32, (1, 2 * D), 1)
        mask_l = (col < D).astype(jnp.float32)
        mask_r = 1.0 - mask_l
        nl = jnp.sum(s * mask_l, axis=-1, keepdims=True)
        nr = jnp.sum(s * mask_r, axis=-1, keepdims=True)
        denom = mask_l * jnp.maximum(jnp.sqrt(nl), 1e-6) + mask_r * jnp.maximum(
            jnp.sqrt(nr), 1e-6
        )
        y2 = r / denom
        t = y2.T
        rT = jnp.concatenate(
            [t[hh * D:(hh + 1) * D, kk * 256:(kk + 1) * 256]
             for kk in range(8) for hh in (0, 1)],
            axis=1,
        )
        for dh in range(8):
            for bb in range(32):
                o_ref[0, dh, bb, :, :] = rT[dh * 8:dh * 8 + 8, bb * 128:(bb + 1) * 128]

    out5 = pl.pallas_call(
        body,
        grid=(n_c, n_k),
        in_specs=[
            pl.BlockSpec((blkj, 2 * D), lambda c, k: (c * n_k + k, 0)),
            pl.BlockSpec((2 * D, 2 * H), lambda c, k: (0, 0)),
            pl.BlockSpec((1, 2 * H), lambda c, k: (0, 0)),
            pl.BlockSpec((2 * H, 2 * D), lambda c, k: (0, 0)),
            pl.BlockSpec((1, 2 * D), lambda c, k: (0, 0)),
        ],
        name="mlp_norm_t",
        out_specs=pl.BlockSpec((1, 8, 32, 8, 128), lambda c, k: (c, 0, k, 0, 0)),
        out_shape=jax.ShapeDtypeStruct((n_c, 8, n_b // 128, 8, 128), jnp.float32),
    )(e2, W1d, b1d, W2d, b2d)
    return out5


def kernel(item_ids, categories, item_table, cat_table, W1, b1, W2, b2):
    B, C = item_ids.shape
    n_rows = B * C
    ids = item_ids.T.reshape(n_rows).astype(jnp.int32)
    cats = categories.T.reshape(n_rows).astype(jnp.int32)
    W1d = (
        jnp.zeros((2 * D, 2 * H), jnp.float32)
        .at[:D, :H].set(W1)
        .at[D:, H:].set(W1)
    )
    W2d = (
        jnp.zeros((2 * H, 2 * D), jnp.float32)
        .at[:H, :D].set(W2)
        .at[H:, D:].set(W2)
    )
    b1d = jnp.concatenate([b1, b1]).reshape(1, 2 * H)
    b2d = jnp.concatenate([b2, b2]).reshape(1, 2 * D)
    e = _sc_gather_add(ids, cats, item_table, cat_table, B)
    e2 = e.reshape(n_rows // 2, 2 * D)
    out5 = _tc_mlp_normalize(
        e2, W1d.astype(jnp.bfloat16), b1d, W2d.astype(jnp.bfloat16), b2d, B, C
    )
    return jnp.transpose(out5, (2, 4, 0, 1, 3)).reshape(B, C, D)


# speedup vs baseline: 1.0056x; 1.0056x over previous
import functools

import jax
import jax.numpy as jnp
from jax import lax
from jax.experimental import pallas as pl
from jax.experimental.pallas import tpu as pltpu
from jax.experimental.pallas import tpu_sc as plsc

D = 64
H = 128
NC, NS = 2, 16
NW = NC * NS
K = 128


def _sc_gather_add(ids, cats, itab, ctab, b_size):
    n_rows = ids.shape[0]
    rows_per_w = n_rows // NW
    n_chunks = rows_per_w // K
    b_mask = b_size - 1
    mesh = plsc.VectorSubcoreMesh(
        core_axis_name="c", subcore_axis_name="s", num_cores=NC, num_subcores=NS
    )

    @functools.partial(
        pl.kernel,
        out_type=jax.ShapeDtypeStruct((n_rows, D), jnp.float32),
        mesh=mesh,
        scratch_types=[
            pltpu.VMEM((2, K), jnp.int32),
            pltpu.VMEM((2, K), jnp.int32),
            pltpu.VMEM((2, K), jnp.int32),
            pltpu.VMEM((2, K, D), jnp.float32),
            pltpu.VMEM((2, K, D), jnp.float32),
            pltpu.VMEM((2, K, D), jnp.float32),
            pltpu.SemaphoreType.DMA,
            pltpu.SemaphoreType.DMA,
            pltpu.SemaphoreType.DMA,
            pltpu.SemaphoreType.DMA,
            pltpu.SemaphoreType.DMA,
            pltpu.SemaphoreType.DMA,
        ],
        compiler_params=pltpu.CompilerParams(use_tc_tiling_on_sc=False),
    )
    def sc_kernel(ids_hbm, cats_hbm, itab_hbm, ctab_hbm, out_hbm,
                  idx_i, idx_c, dst_i, ebuf, cbuf, obuf,
                  sgi0, sgc0, so0, sgi1, sgc1, so1):
        sgi = (sgi0, sgi1)
        sgc = (sgc0, sgc1)
        so = (so0, so1)
        wid = lax.axis_index("s") * NC + lax.axis_index("c")
        base = wid * rows_per_w

        def stage_idx(c, s):
            off = base + c * K
            pltpu.sync_copy(ids_hbm.at[pl.ds(off, K)], idx_i.at[s])
            pltpu.sync_copy(cats_hbm.at[pl.ds(off, K)], idx_c.at[s])

        def issue_gather(s):
            pltpu.async_copy(itab_hbm.at[idx_i.at[s]], ebuf.at[s], sgi[s])
            pltpu.async_copy(ctab_hbm.at[idx_c.at[s]], cbuf.at[s], sgc[s])

        def wait_gather(s):
            pltpu.make_async_copy(itab_hbm.at[idx_i.at[s]], ebuf.at[s], sgi[s]).wait()
            pltpu.make_async_copy(ctab_hbm.at[idx_c.at[s]], cbuf.at[s], sgc[s]).wait()

        def compute_dst(c, s):
            off = base + c * K

            def vec(v, carry):
                pos = off + v * 16 + lax.iota(jnp.int32, 16)
                b = lax.bitwise_and(pos, b_mask)
                dst = (
                    lax.bitwise_and(pos, ~b_mask)
                    + lax.bitwise_and(b, ~511)
                    + lax.shift_left(lax.bitwise_and(b, 255), 1)
                    + lax.bitwise_and(lax.shift_right_logical(b, 8), 1)
                )
                dst_i[s, pl.ds(v * 16, 16)] = dst
                return carry

            lax.fori_loop(0, K // 16, vec, 0)

        def do_add(s):
            def add_row(j, carry):
                for t in range(D // 16):
                    sl = pl.ds(t * 16, 16)
                    obuf[s, j, sl] = ebuf[s, j, sl] + cbuf[s, j, sl]
                return carry
            lax.fori_loop(0, K, add_row, 0)

        def issue_out(s):
            pltpu.async_copy(obuf.at[s], out_hbm.at[dst_i.at[s]], so[s])

        def wait_out(s):
            pltpu.make_async_copy(obuf.at[s], out_hbm.at[dst_i.at[s]], so[s]).wait()

        stage_idx(0, 0)
        issue_gather(0)
        stage_idx(1, 1)
        issue_gather(1)

        def body(g2, carry):
            for s in (0, 1):
                c = 2 * g2 + s
                wait_gather(s)

                @pl.when(g2 > 0)
                def _():
                    wait_out(s)

                compute_dst(c, s)
                do_add(s)
                issue_out(s)

                @pl.when(c + 2 < n_chunks)
                def _():
                    stage_idx(c + 2, s)
                    issue_gather(s)
            return carry

        lax.fori_loop(0, n_chunks // 2, body, 0)
        wait_out(0)
        wait_out(1)

    return sc_kernel(ids, cats, itab, ctab)


def _tc_mlp_normalize(e2, W1d, b1d, W2d, b2d, n_b, n_c):
    blkj = 2048
    n_k = n_b // (2 * blkj)

    def body(e_ref, w1_ref, b1_ref, w2_ref, b2_ref, o_ref):
        ev = e_ref[...]
        h = jnp.dot(ev, w1_ref[...], preferred_element_type=jnp.float32)
        h = jnp.maximum(h + b1_ref[...], 0.0)
        r = jnp.dot(h, w2_ref[...], preferred_element_type=jnp.float32)
        r = r + b2_ref[...] + ev
        s = r * r
        col = lax.broadcasted_iota(jnp.int32, (1, 2 * D), 1)
        mask_l = (col < D).astype(jnp.float32)
        mask_r = 1.0 - mask_l
        nl = jnp.sum(s * mask_l, axis=-1, keepdims=True)
        nr = jnp.sum(s * mask_r, axis=-1, keepdims=True)
        denom = mask_l * jnp.maximum(jnp.sqrt(nl), 1e-6) + mask_r * jnp.maximum(
            jnp.sqrt(nr), 1e-6
        )
        y2 = r / denom
        t = y2.T
        rT = jnp.concatenate(
            [t[hh * D:(hh + 1) * D, kk * 256:(kk + 1) * 256]
             for kk in range(8) for hh in (0, 1)],
            axis=1,
        )
        for dh in range(8):
            for bb in range(32):
                o_ref[0, dh, bb, :, :] = rT[dh * 8:dh * 8 + 8, bb * 128:(bb + 1) * 128]

    out5 = pl.pallas_call(
        body,
        grid=(n_c, n_k),
        in_specs=[
            pl.BlockSpec((blkj, 2 * D), lambda c, k: (c * n_k + k, 0)),
            pl.BlockSpec((2 * D, 2 * H), lambda c, k: (0, 0)),
            pl.BlockSpec((1, 2 * H), lambda c, k: (0, 0)),
            pl.BlockSpec((2 * H, 2 * D), lambda c, k: (0, 0)),
            pl.BlockSpec((1, 2 * D), lambda c, k: (0, 0)),
        ],
        out_specs=pl.BlockSpec((1, 8, 32, 8, 128), lambda c, k: (c, 0, k, 0, 0)),
        out_shape=jax.ShapeDtypeStruct((n_c, 8, n_b // 128, 8, 128), jnp.float32),
    )(e2, W1d, b1d, W2d, b2d)
    return out5


def kernel(item_ids, categories, item_table, cat_table, W1, b1, W2, b2):
    B, C = item_ids.shape
    n_rows = B * C
    ids = item_ids.T.reshape(n_rows).astype(jnp.int32)
    cats = categories.T.reshape(n_rows).astype(jnp.int32)
    W1d = (
        jnp.zeros((2 * D, 2 * H), jnp.float32)
        .at[:D, :H].set(W1)
        .at[D:, H:].set(W1)
    )
    W2d = (
        jnp.zeros((2 * H, 2 * D), jnp.float32)
        .at[:H, :D].set(W2)
        .at[H:, D:].set(W2)
    )
    b1d = jnp.concatenate([b1, b1]).reshape(1, 2 * H)
    b2d = jnp.concatenate([b2, b2]).reshape(1, 2 * D)
    e = _sc_gather_add(ids, cats, item_table, cat_table, B)
    e2 = e.reshape(n_rows // 2, 2 * D)
    out5 = _tc_mlp_normalize(e2, W1d, b1d, W2d, b2d, B, C)
    return jnp.transpose(out5, (2, 4, 0, 1, 3)).reshape(B, C, D)
